# Initial kernel scaffold; baseline (speedup 1.0000x reference)
import jax
import jax.numpy as jnp
from jax.experimental import pallas as pl


def kernel(X, user_mf, movie_mf, user_mlp, movie_mlp, W1, b1, W2, b2, W3, b3, Wf, bf):
    raise NotImplementedError("write your pallas kernel here")


# speedup vs baseline: 5.5973x; 5.5973x over previous
import functools

import jax
import jax.numpy as jnp
from jax import lax
from jax.experimental import pallas as pl
from jax.experimental.pallas import tpu as pltpu
from jax.experimental.pallas import tpu_sc as plsc

BATCH = 16384
MF_DIM = 16
MLP_DIM = 32


def _sc_gather(uidx, midx, user_mf, movie_mf, user_mlp, movie_mlp):
    info = plsc.get_sparse_core_info()
    nc, ns = info.num_cores, info.num_subcores
    nw = nc * ns
    bpw = BATCH // nw
    mesh = plsc.VectorSubcoreMesh(core_axis_name="c", subcore_axis_name="s")

    @functools.partial(
        pl.kernel,
        mesh=mesh,
        out_type=[
            jax.ShapeDtypeStruct((BATCH, MF_DIM), jnp.float32),
            jax.ShapeDtypeStruct((BATCH, MF_DIM), jnp.float32),
            jax.ShapeDtypeStruct((BATCH, MLP_DIM), jnp.float32),
            jax.ShapeDtypeStruct((BATCH, MLP_DIM), jnp.float32),
        ],
        scratch_types=[
            pltpu.VMEM((bpw,), jnp.int32),
            pltpu.VMEM((bpw,), jnp.int32),
            pltpu.VMEM((bpw, MF_DIM), jnp.float32),
            pltpu.VMEM((bpw, MF_DIM), jnp.float32),
            pltpu.VMEM((bpw, MLP_DIM), jnp.float32),
            pltpu.VMEM((bpw, MLP_DIM), jnp.float32),
            pltpu.SemaphoreType.DMA,
            pltpu.SemaphoreType.DMA,
            pltpu.SemaphoreType.DMA,
            pltpu.SemaphoreType.DMA,
        ],
    )
    def k(uidx_h, midx_h, umf_h, mmf_h, umlp_h, mmlp_h,
          umf_o, mmf_o, umlp_o, mmlp_o,
          uidx_v, midx_v, umf_v, mmf_v, umlp_v, mmlp_v,
          s1, s2, s3, s4):
        wid = lax.axis_index("s") * nc + lax.axis_index("c")
        base = wid * bpw
        pltpu.sync_copy(uidx_h.at[pl.ds(base, bpw)], uidx_v)
        pltpu.sync_copy(midx_h.at[pl.ds(base, bpw)], midx_v)
        c1 = pltpu.async_copy(umf_h.at[uidx_v], umf_v, s1)
        c2 = pltpu.async_copy(mmf_h.at[midx_v], mmf_v, s2)
        c3 = pltpu.async_copy(umlp_h.at[uidx_v], umlp_v, s3)
        c4 = pltpu.async_copy(mmlp_h.at[midx_v], mmlp_v, s4)
        c1.wait()
        c2.wait()
        c3.wait()
        c4.wait()
        pltpu.sync_copy(umf_v, umf_o.at[pl.ds(base, bpw)])
        pltpu.sync_copy(mmf_v, mmf_o.at[pl.ds(base, bpw)])
        pltpu.sync_copy(umlp_v, umlp_o.at[pl.ds(base, bpw)])
        pltpu.sync_copy(mmlp_v, mmlp_o.at[pl.ds(base, bpw)])

    return k(uidx, midx, user_mf, movie_mf, user_mlp, movie_mlp)


def _mlp_body(umf_ref, mmf_ref, umlp_ref, mmlp_ref,
              w1u_ref, w1m_ref, b1_ref, w2_ref, b2_ref, w3_ref, b3_ref,
              wfa_ref, wfb_ref, bf_ref, out_ref):
    h1 = jnp.maximum(
        jnp.dot(umlp_ref[...], w1u_ref[...], preferred_element_type=jnp.float32)
        + jnp.dot(mmlp_ref[...], w1m_ref[...], preferred_element_type=jnp.float32)
        + b1_ref[...], 0.0)
    h2 = jnp.maximum(
        jnp.dot(h1, w2_ref[...], preferred_element_type=jnp.float32)
        + b2_ref[...], 0.0)
    h3 = jnp.maximum(
        jnp.dot(h2, w3_ref[...], preferred_element_type=jnp.float32)
        + b3_ref[...], 0.0)
    gmf = umf_ref[...] * mmf_ref[...]
    fin = (jnp.dot(gmf, wfa_ref[...], preferred_element_type=jnp.float32)
           + jnp.dot(h3, wfb_ref[...], preferred_element_type=jnp.float32)
           + bf_ref[0, 0])
    out_ref[...] = 5.0 * jax.nn.sigmoid(fin)


def _tc_mlp(umf, mmf, umlp, mmlp, w1u, w1m, b1, w2t, b2, w3t, b3, wfa, wfb, bf):
    bk = 4096
    grid = (BATCH // bk,)
    full = lambda i: (0, 0)
    row = lambda i: (i, 0)
    return pl.pallas_call(
        _mlp_body,
        grid=grid,
        in_specs=[
            pl.BlockSpec((bk, MF_DIM), row),
            pl.BlockSpec((bk, MF_DIM), row),
            pl.BlockSpec((bk, MLP_DIM), row),
            pl.BlockSpec((bk, MLP_DIM), row),
            pl.BlockSpec((MLP_DIM, 2 * MLP_DIM), full),
            pl.BlockSpec((MLP_DIM, 2 * MLP_DIM), full),
            pl.BlockSpec((1, 2 * MLP_DIM), full),
            pl.BlockSpec((2 * MLP_DIM, 2 * MLP_DIM), full),
            pl.BlockSpec((1, 2 * MLP_DIM), full),
            pl.BlockSpec((2 * MLP_DIM, MLP_DIM), full),
            pl.BlockSpec((1, MLP_DIM), full),
            pl.BlockSpec((MF_DIM, 1), full),
            pl.BlockSpec((MLP_DIM, 1), full),
            pl.BlockSpec((1, 1), full),
        ],
        out_specs=pl.BlockSpec((bk, 1), row),
        out_shape=jax.ShapeDtypeStruct((BATCH, 1), jnp.float32),
    )(umf, mmf, umlp, mmlp, w1u, w1m, b1, w2t, b2, w3t, b3, wfa, wfb, bf)


def kernel(X, user_mf, movie_mf, user_mlp, movie_mlp,
           W1, b1, W2, b2, W3, b3, Wf, bf):
    uidx = X[:, 0]
    midx = X[:, 1]
    umf = jnp.take(user_mf, uidx, axis=0)
    mmf = jnp.take(movie_mf, midx, axis=0)
    umlp = jnp.take(user_mlp, uidx, axis=0)
    mmlp = jnp.take(movie_mlp, midx, axis=0)
    w1t = W1.T
    w1u = w1t[:MLP_DIM, :]
    w1m = w1t[MLP_DIM:, :]
    wft = Wf.T
    wfa = wft[:MF_DIM, :]
    wfb = wft[MF_DIM:, :]
    return _tc_mlp(umf, mmf, umlp, mmlp,
                   w1u, w1m, b1.reshape(1, -1), W2.T, b2.reshape(1, -1),
                   W3.T, b3.reshape(1, -1), wfa, wfb, bf.reshape(1, 1))
